# Initial kernel scaffold; baseline (speedup 1.0000x reference)
import jax
import jax.numpy as jnp
from jax.experimental import pallas as pl


def kernel(idx, msa, pair, nn_g, nn_b, ne_g, ne_b, ex_W, ex_b, ex_g, ex_b2, ee_W, ee_b, ee_g, ee_b2, blk_qW, blk_qb, blk_kW, blk_kb, blk_vW, blk_vb, blk_eW, blk_sW, blk_sb, blk_lng, blk_lnb, blk_lW, blk_lb, l1_W, l1_b, ns_g, ns_b, st_W, st_b, eps):
    raise NotImplementedError("write your pallas kernel here")


# speedup vs baseline: 61.9805x; 61.9805x over previous
import functools

import jax
import jax.numpy as jnp
from jax.experimental import pallas as pl
from jax.experimental.pallas import tpu as pltpu

L = 384
DPAIR = 128
DNODE = 256
DH = 64
HEADS = 4
HC = 256
NBLK = 3
_INV_SQRT_DH = 0.125
_LN_EPS = 1e-5

_ATOMS = ((-0.5272, 1.3593, 0.0), (0.0, 0.0, 0.0), (1.5233, 0.0, 0.0))


def _ln_last(x, g, b):
    mu = jnp.mean(x, axis=-1, keepdims=True)
    xc = x - mu
    var = jnp.mean(xc * xc, axis=-1, keepdims=True)
    return xc * jax.lax.rsqrt(var + _LN_EPS) * g + b


def _pair_embed_kernel(pair_ref, idxr_ref, idxc_ref, ng_ref, nb_ref, w128_ref,
                       wsep_ref, eb_ref, eg_ref, eb2_ref, out_ref):
    pr = pair_ref[...]
    ti, tj, dc = pr.shape
    prn = _ln_last(pr, ng_ref[...].reshape(1, 1, dc), nb_ref[...].reshape(1, 1, dc))
    sep = idxr_ref[...].reshape(1, tj) - idxc_ref[...].reshape(ti, 1)
    mag = jnp.clip(jnp.log(jnp.abs(sep) + 1.0), 0.0, 5.5)
    s = jnp.sign(sep) * mag
    y = jnp.dot(prn.reshape(ti * tj, dc), w128_ref[...],
                preferred_element_type=jnp.float32)
    y = (y.reshape(ti, tj, DH)
         + s[:, :, None] * wsep_ref[...].reshape(1, 1, DH)
         + eb_ref[...].reshape(1, 1, DH))
    out_ref[...] = _ln_last(y, eg_ref[...].reshape(1, 1, DH),
                            eb2_ref[...].reshape(1, 1, DH))


def _blocks_kernel(msa0_ref, nn_g_ref, nn_b_ref, exWT_ref, ex_b_ref, ex_g_ref,
                   ex_b2_ref, pe_ref, qWT_ref, qb_ref, kWT_ref, kb_ref,
                   vWT_ref, vb_ref, eW_ref, sWT_ref, sb_ref, lng_ref, lnb_ref,
                   lWT_ref, lb_ref, x_out_ref, x_cur, x_nxt, kbuf, vbuf, *, tj):
    t = pl.program_id(0)
    j = pl.program_id(1)

    @pl.when(jnp.logical_and(t == 0, j == 0))
    def _():
        m = _ln_last(msa0_ref[...], nn_g_ref[...], nn_b_ref[...])
        x0 = jnp.dot(m, exWT_ref[...], preferred_element_type=jnp.float32) + ex_b_ref[...]
        x_cur[...] = _ln_last(x0, ex_g_ref[...], ex_b2_ref[...])

    @pl.when(jnp.logical_and(t > 0, j == 0))
    def _():
        x_cur[...] = x_nxt[...]

    @pl.when(j == 0)
    def _():
        xc = x_cur[...]
        kbuf[...] = jnp.dot(xc, kWT_ref[0], preferred_element_type=jnp.float32) + kb_ref[0]
        vbuf[...] = jnp.dot(xc, vWT_ref[0], preferred_element_type=jnp.float32) + vb_ref[0]

    x_blk = x_cur[pl.ds(j * tj, tj), :]
    q = jnp.dot(x_blk, qWT_ref[0], preferred_element_type=jnp.float32) + qb_ref[0]
    pe = pe_ref[...]
    k_all = kbuf[...]
    v_all = vbuf[...]
    ii = jax.lax.broadcasted_iota(jnp.int32, (L, tj), 0)
    jjg = jax.lax.broadcasted_iota(jnp.int32, (L, tj), 1) + j * tj
    diag = ii == jjg
    eW = eW_ref[0]
    aggr_parts = []
    for h in range(HEADS):
        sl = slice(h * DH, (h + 1) * DH)
        q_h = q[:, sl]
        k_h = k_all[:, sl]
        v_h = v_all[:, sl]
        eW_h = eW[sl, :]
        qe_h = jnp.dot(q_h, eW_h, preferred_element_type=jnp.float32)
        logits = jax.lax.dot_general(k_h, q_h, (((1,), (1,)), ((), ())),
                                     preferred_element_type=jnp.float32)
        ae = jnp.sum(pe * qe_h[None, :, :], axis=2)
        a = (logits + ae) * _INV_SQRT_DH
        a = jnp.where(diag, -1e30, a)
        m = jnp.max(a, axis=0, keepdims=True)
        ex = jnp.exp(a - m)
        den = jnp.sum(ex, axis=0, keepdims=True)
        att = ex / (den + 1e-16)
        aggr_h = jax.lax.dot_general(att, v_h, (((0,), (0,)), ((), ())),
                                     preferred_element_type=jnp.float32)
        w_h = jnp.sum(att[:, :, None] * pe, axis=0)
        aggr_h = aggr_h + jax.lax.dot_general(w_h, eW_h, (((1,), (1,)), ((), ())),
                                              preferred_element_type=jnp.float32)
        aggr_parts.append(aggr_h)
    aggr = jnp.concatenate(aggr_parts, axis=1)
    out = aggr + jnp.dot(x_blk, sWT_ref[0], preferred_element_type=jnp.float32) + sb_ref[0]
    out = _ln_last(out, lng_ref[0], lnb_ref[0])
    y = jnp.dot(out, lWT_ref[0], preferred_element_type=jnp.float32) + lb_ref[0]
    z = y + x_blk
    xn = jnp.where(z > 0, z, jnp.exp(jnp.minimum(z, 0.0)) - 1.0)
    x_nxt[pl.ds(j * tj, tj), :] = xn
    x_out_ref[...] = xn[None]


def _head_kernel(x_ref, wTt_ref, bt_ref, wTr_ref, br_ref, ns_g_ref, ns_b_ref,
                 stWT_ref, st_b_ref, eps_ref, xyz_ref, state_ref):
    x = x_ref[...]
    eps = eps_ref[0, 0]
    T = 10.0 * (jnp.dot(x, wTt_ref[...], preferred_element_type=jnp.float32) + bt_ref[...])
    R = jnp.dot(x, wTr_ref[...], preferred_element_type=jnp.float32) + br_ref[...]
    rx, ry, rz = R[:, 0:1], R[:, 1:2], R[:, 2:3]
    ang = jnp.sqrt(rx * rx + ry * ry + rz * rz)
    inv = 1.0 / (ang + eps)
    rvx, rvy, rvz = rx * inv, ry * inv, rz * inv
    ca = jnp.cos(ang)
    sa = jnp.sin(ang)
    tx, ty, tz = T[:, 0:1], T[:, 1:2], T[:, 2:3]
    cols = []
    for (vx, vy, vz) in _ATOMS:
        rdv = rvx * vx + rvy * vy + rvz * vz
        cx = rvy * vz - rvz * vy
        cy = rvz * vx - rvx * vz
        cz = rvx * vy - rvy * vx
        px, py, pz = rvx * rdv, rvy * rdv, rvz * rdv
        cols.append((vx - px) * ca + cx * sa + px + tx)
        cols.append((vy - py) * ca + cy * sa + py + ty)
        cols.append((vz - pz) * ca + cz * sa + pz + tz)
    xyz_ref[...] = jnp.concatenate(cols, axis=1)
    xn = _ln_last(x, ns_g_ref[...], ns_b_ref[...])
    state_ref[...] = jnp.dot(xn, stWT_ref[...], preferred_element_type=jnp.float32) + st_b_ref[...]


def kernel(idx, msa, pair, nn_g, nn_b, ne_g, ne_b, ex_W, ex_b, ex_g, ex_b2,
           ee_W, ee_b, ee_g, ee_b2, blk_qW, blk_qb, blk_kW, blk_kb, blk_vW,
           blk_vb, blk_eW, blk_sW, blk_sb, blk_lng, blk_lnb, blk_lW, blk_lb,
           l1_W, l1_b, ns_g, ns_b, st_W, st_b, eps):
    f32 = jnp.float32
    Bb = idx.shape[0]

    TI, TJE = 64, 128
    pair2 = pair.reshape(L, L, DPAIR)
    idxr = idx.astype(f32).reshape(1, L)
    idxc = idx.astype(f32).reshape(L, 1)
    full = lambda shp: pl.BlockSpec(shp, lambda i, j: tuple(0 for _ in shp))
    pe = pl.pallas_call(
        _pair_embed_kernel,
        grid=(L // TI, L // TJE),
        in_specs=[
            pl.BlockSpec((TI, TJE, DPAIR), lambda i, j: (i, j, 0)),
            pl.BlockSpec((1, TJE), lambda i, j: (0, j)),
            pl.BlockSpec((TI, 1), lambda i, j: (i, 0)),
            full((1, DPAIR)), full((1, DPAIR)), full((DPAIR, DH)),
            full((1, DH)), full((1, DH)), full((1, DH)), full((1, DH)),
        ],
        out_specs=pl.BlockSpec((TI, TJE, DH), lambda i, j: (i, j, 0)),
        out_shape=jax.ShapeDtypeStruct((L, L, DH), f32),
    )(pair2, idxr, idxc, ne_g.reshape(1, DPAIR), ne_b.reshape(1, DPAIR),
      ee_W[:, :DPAIR].T, ee_W[:, DPAIR].reshape(1, DH), ee_b.reshape(1, DH),
      ee_g.reshape(1, DH), ee_b2.reshape(1, DH))

    TJ = 64
    NJ = L // TJ
    msa0 = msa[:, 0].reshape(L, DNODE)
    wfull = lambda shp: pl.BlockSpec(shp, lambda t, j: tuple(0 for _ in shp))
    wblk = lambda shp: pl.BlockSpec(shp, lambda t, j: (t,) + tuple(0 for _ in shp[1:]))
    x_fin = pl.pallas_call(
        functools.partial(_blocks_kernel, tj=TJ),
        grid=(NBLK, NJ),
        in_specs=[
            wfull((L, DNODE)), wfull((1, DNODE)), wfull((1, DNODE)),
            wfull((DNODE, DH)), wfull((1, DH)), wfull((1, DH)), wfull((1, DH)),
            pl.BlockSpec((L, TJ, DH), lambda t, j: (0, j, 0)),
            wblk((1, DH, HC)), wblk((1, 1, HC)),
            wblk((1, DH, HC)), wblk((1, 1, HC)),
            wblk((1, DH, HC)), wblk((1, 1, HC)),
            wblk((1, HC, DH)),
            wblk((1, DH, HC)), wblk((1, 1, HC)),
            wblk((1, 1, HC)), wblk((1, 1, HC)),
            wblk((1, HC, DH)), wblk((1, 1, DH)),
        ],
        out_specs=pl.BlockSpec((1, TJ, DH), lambda t, j: (t, j, 0)),
        out_shape=jax.ShapeDtypeStruct((NBLK, L, DH), f32),
        scratch_shapes=[
            pltpu.VMEM((L, DH), f32), pltpu.VMEM((L, DH), f32),
            pltpu.VMEM((L, HC), f32), pltpu.VMEM((L, HC), f32),
        ],
    )(msa0, nn_g.reshape(1, DNODE), nn_b.reshape(1, DNODE), ex_W.T,
      ex_b.reshape(1, DH), ex_g.reshape(1, DH), ex_b2.reshape(1, DH), pe,
      jnp.transpose(blk_qW, (0, 2, 1)), blk_qb.reshape(NBLK, 1, HC),
      jnp.transpose(blk_kW, (0, 2, 1)), blk_kb.reshape(NBLK, 1, HC),
      jnp.transpose(blk_vW, (0, 2, 1)), blk_vb.reshape(NBLK, 1, HC),
      blk_eW,
      jnp.transpose(blk_sW, (0, 2, 1)), blk_sb.reshape(NBLK, 1, HC),
      blk_lng.reshape(NBLK, 1, HC), blk_lnb.reshape(NBLK, 1, HC),
      jnp.transpose(blk_lW, (0, 2, 1)), blk_lb.reshape(NBLK, 1, DH))
    x_fin = x_fin[NBLK - 1]

    xyz9, state = pl.pallas_call(
        _head_kernel,
        out_shape=(jax.ShapeDtypeStruct((L, 9), f32),
                   jax.ShapeDtypeStruct((L, 8), f32)),
    )(x_fin, l1_W[0:3].T, l1_b[0:3].reshape(1, 3), l1_W[3:6].T,
      l1_b[3:6].reshape(1, 3), ns_g.reshape(1, DH), ns_b.reshape(1, DH),
      st_W.T, st_b.reshape(1, 8), jnp.asarray(eps, f32).reshape(1, 1))

    return xyz9.reshape(Bb, L, 3, 3), state.reshape(Bb, L, 8)
